# baseline (speedup 1.0000x reference)
import functools

import jax
import jax.numpy as jnp
from jax import lax
from jax.experimental import pallas as pl
from jax.experimental.pallas import tpu as pltpu
from jax.experimental.pallas import tpu_sc as plsc

_NC = 2
_NS = 16
_NW = _NC * _NS

_IDX_LANES = 128
_DPAD = 64
_VBLK = 512


def _transpose_kernel(x_ref, o_ref):
    x = x_ref[0]
    y = x.T
    z = jnp.concatenate(
        [y, jnp.zeros((y.shape[0], _DPAD - y.shape[1]), jnp.float32)], axis=1
    )
    half = z.shape[0] // 2
    o_ref[0] = jnp.concatenate([z[:half], z[half:]], axis=1)


def _tc_transpose(tables_t: jax.Array) -> jax.Array:
    f, d, v = tables_t.shape
    nvb = (v + _VBLK - 1) // _VBLK
    rows_per_blk = _VBLK * _DPAD // 128
    out_rows = nvb * rows_per_blk
    return pl.pallas_call(
        _transpose_kernel,
        grid=(f, nvb),
        in_specs=[pl.BlockSpec((1, d, _VBLK), lambda fi, vb: (fi, 0, vb))],
        out_specs=pl.BlockSpec((1, rows_per_blk, 128), lambda fi, vb: (fi, vb, 0)),
        out_shape=jax.ShapeDtypeStruct((f, out_rows, 128), jnp.float32),
    )(tables_t)


def _sc_gather(x_cat_t: jax.Array, table64: jax.Array) -> jax.Array:
    f, nb = x_cat_t.shape
    _, vp, dpad = table64.shape
    assert dpad == _DPAD
    per_w_b = nb // _NW
    assert per_w_b * _NW == nb
    k = per_w_b // _IDX_LANES
    assert k * _IDX_LANES == per_w_b
    idx3 = x_cat_t.reshape(f, nb // _IDX_LANES, _IDX_LANES)

    mesh = plsc.VectorSubcoreMesh(
        core_axis_name="c", subcore_axis_name="s", num_cores=_NC, num_subcores=_NS
    )

    @functools.partial(
        pl.kernel,
        mesh=mesh,
        compiler_params=pltpu.CompilerParams(use_tc_tiling_on_sc=False),
        out_type=jax.ShapeDtypeStruct((nb, f * _DPAD), jnp.float32),
        scratch_types=[
            pltpu.VMEM((k, _IDX_LANES), jnp.int32),
            pltpu.VMEM((per_w_b, _DPAD), jnp.float32),
            pltpu.SemaphoreType.DMA,
        ],
    )
    def gather_kernel(idx_hbm, table_hbm, out_hbm, idx_v, rows_v, sem):
        wid = lax.axis_index("s") * _NC + lax.axis_index("c")
        b0 = wid * per_w_b
        brow0 = wid * k

        for fi in range(f):
            pltpu.sync_copy(idx_hbm.at[fi, pl.ds(brow0, k)], idx_v)
            copies = [
                pltpu.async_copy(
                    table_hbm.at[fi].at[idx_v.at[j]],
                    rows_v.at[pl.ds(j * _IDX_LANES, _IDX_LANES)],
                    sem,
                )
                for j in range(k)
            ]
            for cp in copies:
                cp.wait()
            pltpu.sync_copy(
                rows_v, out_hbm.at[pl.ds(b0, per_w_b), pl.ds(fi * _DPAD, _DPAD)]
            )

    return gather_kernel(idx3, table64)


def _mlp_kernel(xn_ref, xe_ref, w1n_ref, w1e_ref, b1_ref, w2_ref, b2_ref,
                wst_ref, bst_ref, out_ref):
    h = jnp.dot(xn_ref[...], w1n_ref[...], preferred_element_type=jnp.float32)
    h += jnp.dot(xe_ref[...], w1e_ref[...], preferred_element_type=jnp.float32)
    h = jnp.maximum(h + b1_ref[...], 0.0)
    g = jnp.dot(h, w2_ref[...], preferred_element_type=jnp.float32)
    g = jnp.maximum(g + b2_ref[...], 0.0)
    out_ref[...] = (
        jnp.dot(g, wst_ref[...], preferred_element_type=jnp.float32) + bst_ref[...]
    )


def _tc_mlp(x_num, x_emb, w1n, w1e, b1, w2, b2, wst, bst, blk):
    b_total, num_f = x_num.shape
    fd = x_emb.shape[1]
    h = w1n.shape[1]
    h2 = w2.shape[1]
    nout = wst.shape[1]
    grid = (b_total // blk,)
    return pl.pallas_call(
        _mlp_kernel,
        grid=grid,
        in_specs=[
            pl.BlockSpec((blk, num_f), lambda i: (i, 0)),
            pl.BlockSpec((blk, fd), lambda i: (i, 0)),
            pl.BlockSpec((num_f, h), lambda i: (0, 0)),
            pl.BlockSpec((fd, h), lambda i: (0, 0)),
            pl.BlockSpec((1, h), lambda i: (0, 0)),
            pl.BlockSpec((h, h2), lambda i: (0, 0)),
            pl.BlockSpec((1, h2), lambda i: (0, 0)),
            pl.BlockSpec((h2, nout), lambda i: (0, 0)),
            pl.BlockSpec((1, nout), lambda i: (0, 0)),
        ],
        out_specs=pl.BlockSpec((blk, nout), lambda i: (i, 0)),
        out_shape=jax.ShapeDtypeStruct((b_total, nout), jnp.float32),
    )(x_num, x_emb, w1n, w1e, b1, w2, b2, wst, bst)


def kernel(x_num, x_cat, tables, W1, b1, W2, b2, Ws, bs, Wt, bt):
    b, f = x_cat.shape
    _, v, d = tables.shape
    num_f = x_num.shape[1]

    tables_t = jnp.transpose(tables, (0, 2, 1))
    packed = _tc_transpose(tables_t)
    table64 = packed.reshape(f, packed.shape[1] * 2, _DPAD)

    vb = x_cat // _VBLK
    r = x_cat % _VBLK
    half = _VBLK // 2
    idx_remap = vb * _VBLK + (r % half) * 2 + (r // half)

    x_emb = _sc_gather(idx_remap.T, table64)

    w1n = W1[:num_f]
    w1e = jnp.pad(
        W1[num_f:].reshape(f, d, -1), ((0, 0), (0, _DPAD - d), (0, 0))
    ).reshape(f * _DPAD, -1)
    wst = jnp.concatenate([Ws, Wt], axis=1)
    bst = jnp.concatenate([bs, bt]).reshape(1, -1)

    out = _tc_mlp(x_num, x_emb, w1n, w1e, b1.reshape(1, -1), W2,
                  b2.reshape(1, -1), wst, bst, blk=512)
    return out[:, :1], out[:, 1:]


# speedup vs baseline: 2.3811x; 2.3811x over previous
import functools

import jax
import jax.numpy as jnp
from jax import lax
from jax.experimental import pallas as pl
from jax.experimental.pallas import tpu as pltpu
from jax.experimental.pallas import tpu_sc as plsc

_NC = 2
_NS = 16
_NW = _NC * _NS

_IDX_LANES = 128
_DPAD = 64
_VBLK = 2048


def _transpose_kernel(x_ref, o_ref):
    x = x_ref[0]
    y = x.T
    z = jnp.concatenate(
        [y, jnp.zeros((y.shape[0], _DPAD - y.shape[1]), jnp.float32)], axis=1
    )
    half = z.shape[0] // 2
    o_ref[0] = jnp.concatenate([z[:half], z[half:]], axis=1)


def _tc_transpose(tables_t: jax.Array) -> jax.Array:
    f, d, v = tables_t.shape
    nvb = (v + _VBLK - 1) // _VBLK
    rows_per_blk = _VBLK * _DPAD // 128
    out_rows = nvb * rows_per_blk
    return pl.pallas_call(
        _transpose_kernel,
        grid=(f, nvb),
        in_specs=[pl.BlockSpec((1, d, _VBLK), lambda fi, vb: (fi, 0, vb))],
        out_specs=pl.BlockSpec((1, rows_per_blk, 128), lambda fi, vb: (fi, vb, 0)),
        out_shape=jax.ShapeDtypeStruct((f, out_rows, 128), jnp.float32),
    )(tables_t)


def _sc_gather(x_cat_t: jax.Array, table64: jax.Array) -> jax.Array:
    f, nb = x_cat_t.shape
    _, vp, dpad = table64.shape
    assert dpad == _DPAD
    per_w_b = nb // _NW
    assert per_w_b * _NW == nb
    k = per_w_b // _IDX_LANES
    assert k * _IDX_LANES == per_w_b
    idx3 = x_cat_t.reshape(f, nb // _IDX_LANES, _IDX_LANES)

    mesh = plsc.VectorSubcoreMesh(
        core_axis_name="c", subcore_axis_name="s", num_cores=_NC, num_subcores=_NS
    )

    @functools.partial(
        pl.kernel,
        mesh=mesh,
        compiler_params=pltpu.CompilerParams(use_tc_tiling_on_sc=False),
        out_type=jax.ShapeDtypeStruct((nb, f * _DPAD), jnp.float32),
        scratch_types=[
            pltpu.VMEM((k, _IDX_LANES), jnp.int32),
            pltpu.VMEM((per_w_b, _DPAD), jnp.float32),
            pltpu.SemaphoreType.DMA,
        ],
    )
    def gather_kernel(idx_hbm, table_hbm, out_hbm, idx_v, rows_v, sem):
        wid = lax.axis_index("s") * _NC + lax.axis_index("c")
        b0 = wid * per_w_b
        brow0 = wid * k

        for fi in range(f):
            pltpu.sync_copy(idx_hbm.at[fi, pl.ds(brow0, k)], idx_v)
            copies = [
                pltpu.async_copy(
                    table_hbm.at[fi].at[idx_v.at[j]],
                    rows_v.at[pl.ds(j * _IDX_LANES, _IDX_LANES)],
                    sem,
                )
                for j in range(k)
            ]
            for cp in copies:
                cp.wait()
            pltpu.sync_copy(
                rows_v, out_hbm.at[pl.ds(b0, per_w_b), pl.ds(fi * _DPAD, _DPAD)]
            )

    return gather_kernel(idx3, table64)


def _mlp_kernel(xn_ref, xe_ref, w1n_ref, w1e_ref, b1_ref, w2_ref, b2_ref,
                wst_ref, bst_ref, out_ref):
    h = jnp.dot(xn_ref[...], w1n_ref[...], preferred_element_type=jnp.float32)
    h += jnp.dot(xe_ref[...], w1e_ref[...], preferred_element_type=jnp.float32)
    h = jnp.maximum(h + b1_ref[...], 0.0)
    g = jnp.dot(h, w2_ref[...], preferred_element_type=jnp.float32)
    g = jnp.maximum(g + b2_ref[...], 0.0)
    out_ref[...] = (
        jnp.dot(g, wst_ref[...], preferred_element_type=jnp.float32) + bst_ref[...]
    )


def _tc_mlp(x_num, x_emb, w1n, w1e, b1, w2, b2, wst, bst, blk):
    b_total, num_f = x_num.shape
    fd = x_emb.shape[1]
    h = w1n.shape[1]
    h2 = w2.shape[1]
    nout = wst.shape[1]
    grid = (b_total // blk,)
    return pl.pallas_call(
        _mlp_kernel,
        grid=grid,
        in_specs=[
            pl.BlockSpec((blk, num_f), lambda i: (i, 0)),
            pl.BlockSpec((blk, fd), lambda i: (i, 0)),
            pl.BlockSpec((num_f, h), lambda i: (0, 0)),
            pl.BlockSpec((fd, h), lambda i: (0, 0)),
            pl.BlockSpec((1, h), lambda i: (0, 0)),
            pl.BlockSpec((h, h2), lambda i: (0, 0)),
            pl.BlockSpec((1, h2), lambda i: (0, 0)),
            pl.BlockSpec((h2, nout), lambda i: (0, 0)),
            pl.BlockSpec((1, nout), lambda i: (0, 0)),
        ],
        out_specs=pl.BlockSpec((blk, nout), lambda i: (i, 0)),
        out_shape=jax.ShapeDtypeStruct((b_total, nout), jnp.float32),
    )(x_num, x_emb, w1n, w1e, b1, w2, b2, wst, bst)


def kernel(x_num, x_cat, tables, W1, b1, W2, b2, Ws, bs, Wt, bt):
    b, f = x_cat.shape
    _, v, d = tables.shape
    num_f = x_num.shape[1]

    tables_t = jnp.transpose(tables, (0, 2, 1))
    packed = _tc_transpose(tables_t)
    table64 = packed.reshape(f, packed.shape[1] * 2, _DPAD)

    vb = x_cat // _VBLK
    r = x_cat % _VBLK
    half = _VBLK // 2
    idx_remap = vb * _VBLK + (r % half) * 2 + (r // half)

    x_emb = _sc_gather(idx_remap.T, table64)

    w1n = W1[:num_f]
    w1e = jnp.pad(
        W1[num_f:].reshape(f, d, -1), ((0, 0), (0, _DPAD - d), (0, 0))
    ).reshape(f * _DPAD, -1)
    wst = jnp.concatenate([Ws, Wt], axis=1)
    bst = jnp.concatenate([bs, bt]).reshape(1, -1)

    out = _tc_mlp(x_num, x_emb, w1n, w1e, b1.reshape(1, -1), W2,
                  b2.reshape(1, -1), wst, bst, blk=512)
    return out[:, :1], out[:, 1:]


# speedup vs baseline: 3.0007x; 1.2602x over previous
import functools

import jax
import jax.numpy as jnp
from jax import lax
from jax.experimental import pallas as pl
from jax.experimental.pallas import tpu as pltpu
from jax.experimental.pallas import tpu_sc as plsc

_NC = 2
_NS = 16
_NW = _NC * _NS

_IDX_LANES = 128
_DPAD = 64
_VBLK = 4096


def _transpose_kernel(x_ref, o_ref):
    x = x_ref[0]
    y = x.T
    z = jnp.concatenate(
        [y, jnp.zeros((y.shape[0], _DPAD - y.shape[1]), jnp.float32)], axis=1
    )
    half = z.shape[0] // 2
    o_ref[0] = jnp.concatenate([z[:half], z[half:]], axis=1)


def _tc_transpose(tables_t: jax.Array) -> jax.Array:
    f, d, v = tables_t.shape
    nvb = (v + _VBLK - 1) // _VBLK
    rows_per_blk = _VBLK * _DPAD // 128
    out_rows = nvb * rows_per_blk
    return pl.pallas_call(
        _transpose_kernel,
        grid=(f, nvb),
        in_specs=[pl.BlockSpec((1, d, _VBLK), lambda fi, vb: (fi, 0, vb))],
        out_specs=pl.BlockSpec((1, rows_per_blk, 128), lambda fi, vb: (fi, vb, 0)),
        out_shape=jax.ShapeDtypeStruct((f, out_rows, 128), jnp.float32),
    )(tables_t)


def _sc_gather(x_cat_t: jax.Array, table64: jax.Array) -> jax.Array:
    f, nb = x_cat_t.shape
    _, vp, dpad = table64.shape
    assert dpad == _DPAD
    per_w_b = nb // _NW
    assert per_w_b * _NW == nb
    k = per_w_b // _IDX_LANES
    assert k * _IDX_LANES == per_w_b
    idx3 = x_cat_t.reshape(f, nb // _IDX_LANES, _IDX_LANES)

    mesh = plsc.VectorSubcoreMesh(
        core_axis_name="c", subcore_axis_name="s", num_cores=_NC, num_subcores=_NS
    )

    @functools.partial(
        pl.kernel,
        mesh=mesh,
        compiler_params=pltpu.CompilerParams(use_tc_tiling_on_sc=False),
        out_type=jax.ShapeDtypeStruct((nb, f * _DPAD), jnp.float32),
        scratch_types=[
            pltpu.VMEM((k, _IDX_LANES), jnp.int32),
            pltpu.VMEM((per_w_b, _DPAD), jnp.float32),
            pltpu.SemaphoreType.DMA,
        ],
    )
    def gather_kernel(idx_hbm, table_hbm, out_hbm, idx_v, rows_v, sem):
        wid = lax.axis_index("s") * _NC + lax.axis_index("c")
        b0 = wid * per_w_b
        brow0 = wid * k

        for fi in range(f):
            pltpu.sync_copy(idx_hbm.at[fi, pl.ds(brow0, k)], idx_v)
            copies = [
                pltpu.async_copy(
                    table_hbm.at[fi].at[idx_v.at[j]],
                    rows_v.at[pl.ds(j * _IDX_LANES, _IDX_LANES)],
                    sem,
                )
                for j in range(k)
            ]
            for cp in copies:
                cp.wait()
            pltpu.sync_copy(
                rows_v, out_hbm.at[pl.ds(b0, per_w_b), pl.ds(fi * _DPAD, _DPAD)]
            )

    return gather_kernel(idx3, table64)


def _mlp_kernel(xn_ref, xe_ref, w1n_ref, w1e_ref, b1_ref, w2_ref, b2_ref,
                wst_ref, bst_ref, out_ref):
    h = jnp.dot(xn_ref[...], w1n_ref[...], preferred_element_type=jnp.float32)
    h += jnp.dot(xe_ref[...], w1e_ref[...], preferred_element_type=jnp.float32)
    h = jnp.maximum(h + b1_ref[...], 0.0)
    g = jnp.dot(h, w2_ref[...], preferred_element_type=jnp.float32)
    g = jnp.maximum(g + b2_ref[...], 0.0)
    out_ref[...] = (
        jnp.dot(g, wst_ref[...], preferred_element_type=jnp.float32) + bst_ref[...]
    )


def _tc_mlp(x_num, x_emb, w1n, w1e, b1, w2, b2, wst, bst, blk):
    b_total, num_f = x_num.shape
    fd = x_emb.shape[1]
    h = w1n.shape[1]
    h2 = w2.shape[1]
    nout = wst.shape[1]
    grid = (b_total // blk,)
    return pl.pallas_call(
        _mlp_kernel,
        grid=grid,
        in_specs=[
            pl.BlockSpec((blk, num_f), lambda i: (i, 0)),
            pl.BlockSpec((blk, fd), lambda i: (i, 0)),
            pl.BlockSpec((num_f, h), lambda i: (0, 0)),
            pl.BlockSpec((fd, h), lambda i: (0, 0)),
            pl.BlockSpec((1, h), lambda i: (0, 0)),
            pl.BlockSpec((h, h2), lambda i: (0, 0)),
            pl.BlockSpec((1, h2), lambda i: (0, 0)),
            pl.BlockSpec((h2, nout), lambda i: (0, 0)),
            pl.BlockSpec((1, nout), lambda i: (0, 0)),
        ],
        out_specs=pl.BlockSpec((blk, nout), lambda i: (i, 0)),
        out_shape=jax.ShapeDtypeStruct((b_total, nout), jnp.float32),
    )(x_num, x_emb, w1n, w1e, b1, w2, b2, wst, bst)


def kernel(x_num, x_cat, tables, W1, b1, W2, b2, Ws, bs, Wt, bt):
    b, f = x_cat.shape
    _, v, d = tables.shape
    num_f = x_num.shape[1]

    tables_t = jnp.transpose(tables, (0, 2, 1))
    packed = _tc_transpose(tables_t)
    table64 = packed.reshape(f, packed.shape[1] * 2, _DPAD)

    vb = x_cat // _VBLK
    r = x_cat % _VBLK
    half = _VBLK // 2
    idx_remap = vb * _VBLK + (r % half) * 2 + (r // half)

    x_emb = _sc_gather(idx_remap.T, table64)

    w1n = W1[:num_f]
    w1e = jnp.pad(
        W1[num_f:].reshape(f, d, -1), ((0, 0), (0, _DPAD - d), (0, 0))
    ).reshape(f * _DPAD, -1)
    wst = jnp.concatenate([Ws, Wt], axis=1)
    bst = jnp.concatenate([bs, bt]).reshape(1, -1)

    out = _tc_mlp(x_num, x_emb, w1n, w1e, b1.reshape(1, -1), W2,
                  b2.reshape(1, -1), wst, bst, blk=512)
    return out[:, :1], out[:, 1:]
